# baseline (device time: 14791 ns/iter reference)
import jax
import jax.numpy as jnp
from jax import lax
from jax.experimental import pallas as pl
from jax.experimental.pallas import tpu as pltpu

NC = 16


def kernel(x, dest):
    t, d = x.shape
    c = t // NC

    def body(
        x_ref,
        dest_ref,
        out_ref,
        xs,
        xloc,
        px,
        xf,
        dvm,
        send_sems,
        recv_sems,
        in_sems,
    ):
        p = lax.axis_index("y")
        mx = lax.axis_index("x")
        mz = lax.axis_index("z")
        peer = (mx, 1 - p, mz)

        bar = pltpu.get_barrier_semaphore()
        pl.semaphore_signal(
            bar, inc=1, device_id=peer, device_id_type=pl.DeviceIdType.MESH
        )

        cp_x = pltpu.make_async_copy(x_ref, xf, in_sems.at[0])
        cp_d = pltpu.make_async_copy(dest_ref, dvm, in_sems.at[1])
        cp_x.start()
        cp_d.start()

        cp_d.wait()
        d2 = dvm[...].reshape(1, t)
        md = d2 == p
        mdi = md.astype(jnp.int32)
        mdf = md.astype(jnp.float32)
        tri = (
            lax.broadcasted_iota(jnp.int32, (t, t), 0)
            <= lax.broadcasted_iota(jnp.int32, (t, t), 1)
        ).astype(jnp.float32)
        v = jnp.dot(mdf, tri, preferred_element_type=jnp.float32).astype(
            jnp.int32
        )
        j_vec = lax.broadcasted_iota(jnp.int32, (1, t), 1)
        n_self = jnp.sum(mdi)
        m = t - n_self
        c_keep = v - mdi
        c_send = j_vec - v + mdi
        rank = jnp.where(md, m + c_keep, c_send)

        cp_x.wait()
        xs[...] = xf[...].astype(jnp.bfloat16)

        pl.semaphore_wait(bar, 1)

        rdmas = []
        for k in range(NC):
            rk = pltpu.make_async_remote_copy(
                src_ref=xloc.at[pl.ds(k * c, c)],
                dst_ref=px.at[pl.ds(k * c, c)],
                send_sem=send_sems.at[k],
                recv_sem=recv_sems.at[k],
                device_id=peer,
                device_id_type=pl.DeviceIdType.MESH,
            )
            rdmas.append(rk)
            i_io = lax.broadcasted_iota(jnp.int32, (c, t), 0) + k * c
            p_k = (i_io == rank).astype(jnp.bfloat16)
            xloc[pl.ds(k * c, c), :] = jnp.dot(
                p_k, xs[...], preferred_element_type=jnp.float32
            ).astype(jnp.bfloat16)

            @pl.when(k * c < m)
            def _(rk=rk):
                rk.start()

        shift = (1 - p) * n_self
        row_i = lax.broadcasted_iota(jnp.int32, (t, 1), 0)
        zero = jnp.array(0, jnp.bfloat16)
        keep_rolled = pltpu.roll(
            jnp.where(row_i < m, zero, xloc[...]), shift, 0
        )

        for k in range(NC):
            @pl.when(k * c < m)
            def _(k=k):
                rdmas[k].wait_recv()

        recv_rolled = pltpu.roll(
            jnp.where(row_i < m, px[...], zero), shift, 0
        )
        out_ref[...] = keep_rolled + recv_rolled

        for k in range(NC):
            @pl.when(k * c < m)
            def _(k=k):
                rdmas[k].wait_send()

    return pl.pallas_call(
        body,
        out_shape=jax.ShapeDtypeStruct((t, d), jnp.bfloat16),
        in_specs=[
            pl.BlockSpec(memory_space=pltpu.MemorySpace.HBM),
            pl.BlockSpec(memory_space=pltpu.MemorySpace.HBM),
        ],
        out_specs=pl.BlockSpec(memory_space=pltpu.VMEM),
        scratch_shapes=[
            pltpu.VMEM((t, d), jnp.bfloat16),
            pltpu.VMEM((t, d), jnp.bfloat16),
            pltpu.VMEM((t, d), jnp.bfloat16),
            pltpu.VMEM((t, d), jnp.float32),
            pltpu.VMEM((t,), jnp.int32),
            pltpu.SemaphoreType.DMA((NC,)),
            pltpu.SemaphoreType.DMA((NC,)),
            pltpu.SemaphoreType.DMA((2,)),
        ],
        compiler_params=pltpu.CompilerParams(collective_id=0),
    )(x, dest)


# device time: 14513 ns/iter; 1.0192x vs baseline; 1.0192x over previous
import jax
import jax.numpy as jnp
from jax import lax
from jax.experimental import pallas as pl
from jax.experimental.pallas import tpu as pltpu

NC = 16


def kernel(x, dest):
    t, d = x.shape
    c = t // NC
    dest2 = dest.reshape(1, t).astype(jnp.int32)

    def body(x_ref, dest_ref, out_ref, xs, xsend, px, send_sems, recv_sems):
        p = lax.axis_index("y")
        mx = lax.axis_index("x")
        mz = lax.axis_index("z")
        peer = (mx, 1 - p, mz)

        xs[...] = x_ref[...].astype(jnp.bfloat16)

        md = dest_ref[...] == p
        mdi = md.astype(jnp.int32)
        mdf = md.astype(jnp.float32)
        tri = (
            lax.broadcasted_iota(jnp.int32, (t, t), 0)
            <= lax.broadcasted_iota(jnp.int32, (t, t), 1)
        ).astype(jnp.float32)
        ck_incl = jnp.dot(mdf, tri, preferred_element_type=jnp.float32)
        ck_incl_i = ck_incl.astype(jnp.int32)
        j_vec = lax.broadcasted_iota(jnp.int32, (1, t), 1)
        rank_keep_local = ck_incl_i - mdi
        rank_send = j_vec - ck_incl_i + mdi

        n_self = jnp.sum(mdi)
        m = t - n_self

        bar = pltpu.get_barrier_semaphore()
        pl.semaphore_signal(
            bar, inc=1, device_id=peer, device_id_type=pl.DeviceIdType.MESH
        )
        pl.semaphore_wait(bar, 1)

        rdmas = []
        for k in range(NC):
            rk = pltpu.make_async_remote_copy(
                src_ref=xsend.at[pl.ds(k * c, c)],
                dst_ref=px.at[pl.ds(k * c, c)],
                send_sem=send_sems.at[k],
                recv_sem=recv_sems.at[k],
                device_id=peer,
                device_id_type=pl.DeviceIdType.MESH,
            )
            rdmas.append(rk)

            @pl.when(k * c < m)
            def _(k=k, rk=rk):
                i_io = lax.broadcasted_iota(jnp.int32, (c, t), 0) + k * c
                s_k = ((i_io == rank_send) & (~md)).astype(jnp.bfloat16)
                xsend[pl.ds(k * c, c), :] = jnp.dot(
                    s_k, xs[...], preferred_element_type=jnp.float32
                ).astype(jnp.bfloat16)
                rk.start()

        o_self = p * m
        rank_keep = rank_keep_local + o_self
        i_iota = lax.broadcasted_iota(jnp.int32, (t, t), 0)
        kmat = ((i_iota == rank_keep) & md).astype(jnp.bfloat16)
        own = jnp.dot(kmat, xs[...], preferred_element_type=jnp.float32).astype(
            jnp.bfloat16
        )

        for k in range(NC):
            @pl.when(k * c < m)
            def _(k=k):
                rdmas[k].wait_recv()

        o_in = (1 - p) * n_self
        row_i = lax.broadcasted_iota(jnp.int32, (t, 1), 0)
        pxm = jnp.where(row_i < m, px[...], jnp.array(0, jnp.bfloat16))
        out_ref[...] = own + pltpu.roll(pxm, o_in, 0)

        for k in range(NC):
            @pl.when(k * c < m)
            def _(k=k):
                rdmas[k].wait_send()

    return pl.pallas_call(
        body,
        out_shape=jax.ShapeDtypeStruct((t, d), jnp.bfloat16),
        in_specs=[
            pl.BlockSpec(memory_space=pltpu.VMEM),
            pl.BlockSpec(memory_space=pltpu.VMEM),
        ],
        out_specs=pl.BlockSpec(memory_space=pltpu.VMEM),
        scratch_shapes=[
            pltpu.VMEM((t, d), jnp.bfloat16),
            pltpu.VMEM((t, d), jnp.bfloat16),
            pltpu.VMEM((t, d), jnp.bfloat16),
            pltpu.SemaphoreType.DMA((NC + 1,)),
            pltpu.SemaphoreType.DMA((NC + 1,)),
        ],
        compiler_params=pltpu.CompilerParams(collective_id=0),
    )(x, dest2)
